# baseline (speedup 1.0000x reference)
import jax
import jax.numpy as jnp
from jax import lax
from jax.experimental import pallas as pl
from jax.experimental.pallas import tpu as pltpu
from jax.experimental.pallas import tpu_sc as plsc

_N, _M, _P = 10000, 16, 16
_MP = _M * _P
_R = _N * _M
_GW = 128
_BR = 2000
_NB = _R // _BR
_EB = 2000


def _pre_body(e_ref, bd2_ref, wb_ref, v_ref, b_ref):
    x = e_ref[...]
    v_ref[...] = lax.dot(x, bd2_ref[...], precision=lax.Precision.HIGHEST,
                         preferred_element_type=jnp.float32)
    b_ref[...] = lax.dot(x, wb_ref[...], precision=lax.Precision.HIGHEST,
                         preferred_element_type=jnp.float32)


def _main_body(af_ref, gv_ref, gb_ref, e3_ref, waf_ref, we2_ref, ones_ref,
               cst_ref, out_ref):
    af = af_ref[...]
    xaf = jnp.dot(af, waf_ref[...], preferred_element_type=jnp.float32)
    x2 = jnp.concatenate([e3_ref[...], gb_ref[...]], axis=1)
    xe = jnp.dot(x2, we2_ref[...], preferred_element_type=jnp.float32)
    lin = xaf[:, :_MP] + xe[:, :_MP] + gv_ref[...] + cst_ref[0:1, :]
    att = xaf[:, _MP:] + xe[:, _MP:] + cst_ref[1:2, :]
    alpha = jnp.where(att >= 0, att, 0.01 * att)
    s = af + alpha * lin
    ones16 = ones_ref[...]
    mu = jnp.dot(s, ones16, preferred_element_type=jnp.float32)
    d = s - mu
    var = jnp.dot(d * d, ones16, preferred_element_type=jnp.float32)
    xhat = d * lax.rsqrt(var + 1e-5)
    y = xhat * cst_ref[2:3, :] + cst_ref[3:4, :]
    out_ref[...] = jnp.maximum(y, 0.0) + jnp.log(1.0 + jnp.exp(-jnp.abs(y)))


def _sc_gather(v_table, b_table, idx2):
    mesh = plsc.VectorSubcoreMesh(core_axis_name="core",
                                  subcore_axis_name="subcore")

    @pl.kernel(
        out_type=[jax.ShapeDtypeStruct((_R, _MP), jnp.float32),
                  jax.ShapeDtypeStruct((_R, _M), jnp.float32)],
        mesh=mesh,
        compiler_params=pltpu.CompilerParams(use_tc_tiling_on_sc=False))
    def gather_kernel(v_hbm, b_hbm, i_hbm, gv_hbm, gb_hbm):
        def body(i_vmem, gv_vmem, gb_vmem):
            pltpu.sync_copy(v_hbm.at[i_vmem.at[0]], gv_vmem)
            pltpu.sync_copy(b_hbm.at[i_vmem.at[0]], gb_vmem)

        pltpu.emit_pipeline(
            body,
            grid=(_R // _GW,),
            in_specs=[pl.BlockSpec((1, _GW), lambda i: (0, i))],
            out_specs=[pl.BlockSpec((_GW, _MP), lambda i: (i, 0)),
                       pl.BlockSpec((_GW, _M), lambda i: (i, 0))],
            core_axis_name=("core", "subcore"),
            dimension_semantics=(pltpu.PARALLEL,),
        )(i_hbm, gv_hbm, gb_hbm)

    return gather_kernel(v_table, b_table, idx2)


def kernel(angle_fea, edge_fea, nbr_idx, W_att, b_att, W_lin, b_lin, gamma,
           beta):
    f32 = jnp.float32
    eye = jnp.eye(_M, dtype=f32)
    W1, W2, W3 = W_lin[:_P], W_lin[_P:2 * _P], W_lin[2 * _P:]
    w1, w2, w3 = W_att[:_P, 0], W_att[_P:2 * _P, 0], W_att[2 * _P:, 0]

    bd2 = jnp.kron(eye, W2)
    wb = jnp.kron(eye, w2[:, None])
    bd3 = jnp.kron(eye, W3)
    bdv3 = jnp.kron(eye, jnp.tile(w3[:, None], (1, _P)))
    waf = jnp.concatenate([bd3, bdv3], axis=1)
    w1t = jnp.tile(W1, (1, _M))
    wa1 = jnp.tile(w1[:, None], (1, _MP))
    rep = jnp.kron(eye, jnp.ones((1, _P), f32))
    we2 = jnp.concatenate(
        [jnp.concatenate([w1t, wa1], axis=1),
         jnp.concatenate([jnp.zeros((_M, _MP), f32), rep], axis=1)], axis=0)
    ones16 = jnp.kron(eye, jnp.full((_P, _P), 1.0 / _P, f32))
    consts = jnp.stack([
        jnp.tile(b_lin, _M),
        jnp.full((_MP,), b_att[0], f32),
        jnp.tile(gamma, _M),
        jnp.tile(beta, _M),
    ])

    edge2 = edge_fea.reshape(_N, _MP)
    af2 = angle_fea.reshape(_R, _MP)
    e3 = edge_fea.reshape(_R, _P)
    idx2 = nbr_idx.reshape(1, _R)

    v_table, b_table = pl.pallas_call(
        _pre_body,
        grid=(_N // _EB,),
        in_specs=[
            pl.BlockSpec((_EB, _MP), lambda i: (i, 0)),
            pl.BlockSpec((_MP, _MP), lambda i: (0, 0)),
            pl.BlockSpec((_MP, _M), lambda i: (0, 0)),
        ],
        out_specs=[
            pl.BlockSpec((_EB, _MP), lambda i: (i, 0)),
            pl.BlockSpec((_EB, _M), lambda i: (i, 0)),
        ],
        out_shape=[jax.ShapeDtypeStruct((_N, _MP), f32),
                   jax.ShapeDtypeStruct((_N, _M), f32)],
        compiler_params=pltpu.CompilerParams(
            dimension_semantics=("parallel",)),
    )(edge2, bd2, wb)

    gv, gb = _sc_gather(v_table, b_table, idx2)

    out = pl.pallas_call(
        _main_body,
        grid=(_NB,),
        in_specs=[
            pl.BlockSpec((_BR, _MP), lambda i: (i, 0)),
            pl.BlockSpec((_BR, _MP), lambda i: (i, 0)),
            pl.BlockSpec((_BR, _M), lambda i: (i, 0)),
            pl.BlockSpec((_BR, _P), lambda i: (i, 0)),
            pl.BlockSpec((_MP, 2 * _MP), lambda i: (0, 0)),
            pl.BlockSpec((2 * _M, 2 * _MP), lambda i: (0, 0)),
            pl.BlockSpec((_MP, _MP), lambda i: (0, 0)),
            pl.BlockSpec((4, _MP), lambda i: (0, 0)),
        ],
        out_specs=pl.BlockSpec((_BR, _MP), lambda i: (i, 0)),
        out_shape=jax.ShapeDtypeStruct((_R, _MP), f32),
        compiler_params=pltpu.CompilerParams(
            dimension_semantics=("parallel",)),
    )(af2, gv, gb, e3, waf, we2, ones16, consts)

    return out.reshape(_N, _M, _M, _P)


# speedup vs baseline: 1.0890x; 1.0890x over previous
import jax
import jax.numpy as jnp
from jax import lax
from jax.experimental import pallas as pl
from jax.experimental.pallas import tpu as pltpu
from jax.experimental.pallas import tpu_sc as plsc

_N, _M, _P = 10000, 16, 16
_MP = _M * _P
_R = _N * _M
_GW = 128
_BR = 2000
_NB = _R // _BR


def _main_body(af_ref, ge_ref, e3_ref, waf_ref, wg_ref, we2_ref, ones_ref,
               cst_ref, out_ref):
    af = af_ref[...]
    xaf = jnp.dot(af, waf_ref[...], preferred_element_type=jnp.float32)
    xg = jnp.dot(ge_ref[...], wg_ref[...], preferred_element_type=jnp.float32)
    xe = jnp.dot(e3_ref[...], we2_ref[...], preferred_element_type=jnp.float32)
    lin = xaf[:, :_MP] + xg[:, :_MP] + xe[:, :_MP] + cst_ref[0:1, :]
    att = xaf[:, _MP:] + xg[:, _MP:] + xe[:, _MP:] + cst_ref[1:2, :]
    alpha = jnp.where(att >= 0, att, 0.01 * att)
    s = af + alpha * lin
    ones16 = ones_ref[...]
    mu = jnp.dot(s, ones16, preferred_element_type=jnp.float32)
    d = s - mu
    var = jnp.dot(d * d, ones16, preferred_element_type=jnp.float32)
    xhat = d * lax.rsqrt(var + 1e-5)
    y = xhat * cst_ref[2:3, :] + cst_ref[3:4, :]
    out_ref[...] = jnp.maximum(y, 0.0) + jnp.log(1.0 + jnp.exp(-jnp.abs(y)))


def _sc_gather(table, idx2):
    mesh = plsc.VectorSubcoreMesh(core_axis_name="core",
                                  subcore_axis_name="subcore")

    @pl.kernel(
        out_type=jax.ShapeDtypeStruct((_R, _MP), jnp.float32),
        mesh=mesh)
    def gather_kernel(t_hbm, i_hbm, g_hbm):
        def body(i_vmem, g_vmem):
            pltpu.sync_copy(t_hbm.at[i_vmem.at[0]], g_vmem)

        pltpu.emit_pipeline(
            body,
            grid=(_R // _GW,),
            in_specs=[pl.BlockSpec((1, _GW), lambda i: (0, i))],
            out_specs=[pl.BlockSpec((_GW, _MP), lambda i: (i, 0))],
            core_axis_name=("core", "subcore"),
            dimension_semantics=(pltpu.PARALLEL,),
        )(i_hbm, g_hbm)

    return gather_kernel(table, idx2)


def kernel(angle_fea, edge_fea, nbr_idx, W_att, b_att, W_lin, b_lin, gamma,
           beta):
    f32 = jnp.float32
    eye = jnp.eye(_M, dtype=f32)
    W1, W2, W3 = W_lin[:_P], W_lin[_P:2 * _P], W_lin[2 * _P:]
    w1, w2, w3 = W_att[:_P, 0], W_att[_P:2 * _P, 0], W_att[2 * _P:, 0]

    bd3 = jnp.kron(eye, W3)
    bdv3 = jnp.kron(eye, jnp.tile(w3[:, None], (1, _P)))
    waf = jnp.concatenate([bd3, bdv3], axis=1)
    bd2 = jnp.kron(eye, W2)
    bdv2 = jnp.kron(eye, jnp.tile(w2[:, None], (1, _P)))
    wg = jnp.concatenate([bd2, bdv2], axis=1)
    w1t = jnp.tile(W1, (1, _M))
    wa1 = jnp.tile(w1[:, None], (1, _MP))
    we2 = jnp.concatenate([w1t, wa1], axis=1)
    ones16 = jnp.kron(eye, jnp.full((_P, _P), 1.0 / _P, f32))
    consts = jnp.stack([
        jnp.tile(b_lin, _M),
        jnp.full((_MP,), b_att[0], f32),
        jnp.tile(gamma, _M),
        jnp.tile(beta, _M),
    ])

    edge2 = edge_fea.reshape(_N, _MP)
    af2 = angle_fea.reshape(_R, _MP)
    e3 = edge_fea.reshape(_R, _P)
    idx2 = nbr_idx.reshape(1, _R)

    ge = _sc_gather(edge2, idx2)

    out = pl.pallas_call(
        _main_body,
        grid=(_NB,),
        in_specs=[
            pl.BlockSpec((_BR, _MP), lambda i: (i, 0)),
            pl.BlockSpec((_BR, _MP), lambda i: (i, 0)),
            pl.BlockSpec((_BR, _P), lambda i: (i, 0)),
            pl.BlockSpec((_MP, 2 * _MP), lambda i: (0, 0)),
            pl.BlockSpec((_MP, 2 * _MP), lambda i: (0, 0)),
            pl.BlockSpec((_M, 2 * _MP), lambda i: (0, 0)),
            pl.BlockSpec((_MP, _MP), lambda i: (0, 0)),
            pl.BlockSpec((4, _MP), lambda i: (0, 0)),
        ],
        out_specs=pl.BlockSpec((_BR, _MP), lambda i: (i, 0)),
        out_shape=jax.ShapeDtypeStruct((_R, _MP), f32),
        compiler_params=pltpu.CompilerParams(
            dimension_semantics=("parallel",)),
    )(af2, ge, e3, waf, wg, we2, ones16, consts)

    return out.reshape(_N, _M, _M, _P)


# speedup vs baseline: 7.8181x; 7.1794x over previous
import jax
import jax.numpy as jnp
from jax import lax
from jax.experimental import pallas as pl
from jax.experimental.pallas import tpu as pltpu
from jax.experimental.pallas import tpu_sc as plsc

_N, _M, _P = 10000, 16, 16
_MP = _M * _P
_R = _N * _M
_GW = 128
_BN = 128
_NB = -(-_N // _BN)
_EB = 2048


def _tr_edge_body(et_ref, e2_ref):
    s = et_ref.shape
    x = et_ref[...].reshape(s[0] * s[1], s[2])
    e2_ref[...] = jnp.transpose(x)


def _main_body(at_ref, ge_ref, e2_ref, waf_ref, wg_ref, w1t_ref, ones_ref,
               cst_ref, out_ref):
    bn = at_ref.shape[3]
    m = at_ref.shape[0]
    mp = at_ref.shape[1] * at_ref.shape[2]
    a2 = at_ref[...].reshape(m, mp, bn)
    af = jnp.concatenate([jnp.transpose(a2[j]) for j in range(m)], axis=0)
    ge = ge_ref[...].reshape(m * bn, mp)
    ed = e2_ref[...]
    e3 = jnp.concatenate([ed[:, 16 * j:16 * j + 16] for j in range(m)],
                         axis=0)

    xaf = jnp.dot(af, waf_ref[...], preferred_element_type=jnp.float32)
    xg = jnp.dot(ge, wg_ref[...], preferred_element_type=jnp.float32)
    ut = jnp.dot(e3, w1t_ref[...], preferred_element_type=jnp.float32)
    a_col = jnp.sum(e3 * cst_ref[4:5, :16], axis=1, keepdims=True)

    lin = xaf[:, :mp] + xg[:, :mp] + ut + cst_ref[0:1, :]
    att = xaf[:, mp:] + xg[:, mp:] + a_col + cst_ref[1:2, :]
    alpha = jnp.where(att >= 0, att, 0.01 * att)
    s = af + alpha * lin
    ones16 = ones_ref[...]
    mu = jnp.dot(s, ones16, preferred_element_type=jnp.float32)
    d = s - mu
    var = jnp.dot(d * d, ones16, preferred_element_type=jnp.float32)
    xhat = d * lax.rsqrt(var + 1e-5)
    y = xhat * cst_ref[2:3, :] + cst_ref[3:4, :]
    o2 = jnp.maximum(y, 0.0) + jnp.log(1.0 + jnp.exp(-jnp.abs(y)))
    ot = jnp.stack([jnp.transpose(o2[j * bn:(j + 1) * bn]) for j in range(m)],
                   axis=0)
    out_ref[...] = ot.reshape(m, at_ref.shape[1], at_ref.shape[2], bn)


def _sc_gather(table, idx2):
    mesh = plsc.VectorSubcoreMesh(core_axis_name="core",
                                  subcore_axis_name="subcore")

    @pl.kernel(
        out_type=jax.ShapeDtypeStruct((_R, _MP), jnp.float32),
        mesh=mesh)
    def gather_kernel(t_hbm, i_hbm, g_hbm):
        def body(i_vmem, g_vmem):
            pltpu.sync_copy(t_hbm.at[i_vmem.at[0]], g_vmem)

        pltpu.emit_pipeline(
            body,
            grid=(_R // _GW,),
            in_specs=[pl.BlockSpec((1, _GW), lambda i: (0, i))],
            out_specs=[pl.BlockSpec((_GW, _MP), lambda i: (i, 0))],
            core_axis_name=("core", "subcore"),
            dimension_semantics=(pltpu.PARALLEL,),
        )(i_hbm, g_hbm)

    return gather_kernel(table, idx2)


def kernel(angle_fea, edge_fea, nbr_idx, W_att, b_att, W_lin, b_lin, gamma,
           beta):
    f32 = jnp.float32
    eye = jnp.eye(_M, dtype=f32)
    W1, W2, W3 = W_lin[:_P], W_lin[_P:2 * _P], W_lin[2 * _P:]
    w1, w2, w3 = W_att[:_P, 0], W_att[_P:2 * _P, 0], W_att[2 * _P:, 0]

    bd3 = jnp.kron(eye, W3)
    bdv3 = jnp.kron(eye, jnp.tile(w3[:, None], (1, _P)))
    waf = jnp.concatenate([bd3, bdv3], axis=1)
    bd2 = jnp.kron(eye, W2)
    bdv2 = jnp.kron(eye, jnp.tile(w2[:, None], (1, _P)))
    wg = jnp.concatenate([bd2, bdv2], axis=1)
    w1t = jnp.tile(W1, (1, _M))
    ones16 = jnp.kron(eye, jnp.full((_P, _P), 1.0 / _P, f32))
    consts = jnp.stack([
        jnp.tile(b_lin, _M),
        jnp.full((_MP,), b_att[0], f32),
        jnp.tile(gamma, _M),
        jnp.tile(beta, _M),
        jnp.concatenate([w1, jnp.zeros((_MP - _P,), f32)]),
    ])

    angle_t = jnp.transpose(angle_fea, (1, 2, 3, 0))
    edge_t = jnp.transpose(edge_fea, (1, 2, 0))
    idx2 = jnp.transpose(nbr_idx, (1, 0)).reshape(1, _R)

    edge2 = pl.pallas_call(
        _tr_edge_body,
        grid=(-(-_N // _EB),),
        in_specs=[pl.BlockSpec((_M, _P, _EB), lambda i: (0, 0, i))],
        out_specs=pl.BlockSpec((_EB, _MP), lambda i: (i, 0)),
        out_shape=jax.ShapeDtypeStruct((_N, _MP), f32),
        compiler_params=pltpu.CompilerParams(
            dimension_semantics=("parallel",)),
    )(edge_t)

    ge = _sc_gather(edge2, idx2)
    ge3 = ge.reshape(_M, _N, _MP)

    out_t = pl.pallas_call(
        _main_body,
        grid=(_NB,),
        in_specs=[
            pl.BlockSpec((_M, _P, _P, _BN), lambda i: (0, 0, 0, i)),
            pl.BlockSpec((_M, _BN, _MP), lambda i: (0, i, 0)),
            pl.BlockSpec((_BN, _MP), lambda i: (i, 0)),
            pl.BlockSpec((_MP, 2 * _MP), lambda i: (0, 0)),
            pl.BlockSpec((_MP, 2 * _MP), lambda i: (0, 0)),
            pl.BlockSpec((_P, _MP), lambda i: (0, 0)),
            pl.BlockSpec((_MP, _MP), lambda i: (0, 0)),
            pl.BlockSpec((5, _MP), lambda i: (0, 0)),
        ],
        out_specs=pl.BlockSpec((_M, _P, _P, _BN), lambda i: (0, 0, 0, i)),
        out_shape=jax.ShapeDtypeStruct((_M, _P, _P, _N), f32),
        compiler_params=pltpu.CompilerParams(
            dimension_semantics=("parallel",)),
    )(angle_t, ge3, edge2, waf, wg, w1t, ones16, consts)

    return jnp.transpose(out_t, (3, 0, 1, 2))


# speedup vs baseline: 8.2339x; 1.0532x over previous
import functools

import jax
import jax.numpy as jnp
from jax import lax
from jax.experimental import pallas as pl
from jax.experimental.pallas import tpu as pltpu
from jax.experimental.pallas import tpu_sc as plsc

_N, _M, _P = 10000, 16, 16
_MP = _M * _P
_R = _N * _M
_GW = 128
_NC = 4
_JC = _M // _NC
_BN = 512
_NB = -(-_N // _BN)
_EB = 2048


def _tr_edge_body(et_ref, e2_ref):
    s = et_ref.shape
    x = et_ref[...].reshape(s[0] * s[1], s[2])
    e2_ref[...] = jnp.transpose(x)


def _main_body(at_ref, ge_ref, e2_ref, waf_ref, wg_ref, w1t_ref, ones_ref,
               cst_ref, out_ref, *, j_off):
    bn = at_ref.shape[3]
    m = at_ref.shape[0]
    mp = at_ref.shape[1] * at_ref.shape[2]
    a2 = at_ref[...].reshape(m, mp, bn)
    af = jnp.concatenate([jnp.transpose(a2[j]) for j in range(m)], axis=0)
    ge = ge_ref[...].reshape(m * bn, mp)
    ed = e2_ref[...]
    e3 = jnp.concatenate(
        [ed[:, 16 * (j_off + j):16 * (j_off + j) + 16] for j in range(m)],
        axis=0)

    xaf = jnp.dot(af, waf_ref[...], preferred_element_type=jnp.float32)
    xg = jnp.dot(ge, wg_ref[...], preferred_element_type=jnp.float32)
    ut = jnp.dot(e3, w1t_ref[...], preferred_element_type=jnp.float32)
    a_col = jnp.sum(e3 * cst_ref[4:5, :16], axis=1, keepdims=True)

    lin = xaf[:, :mp] + xg[:, :mp] + ut + cst_ref[0:1, :]
    att = xaf[:, mp:] + xg[:, mp:] + a_col + cst_ref[1:2, :]
    alpha = jnp.where(att >= 0, att, 0.01 * att)
    s = af + alpha * lin
    ones16 = ones_ref[...]
    mu = jnp.dot(s, ones16, preferred_element_type=jnp.float32)
    d = s - mu
    var = jnp.dot(d * d, ones16, preferred_element_type=jnp.float32)
    xhat = d * lax.rsqrt(var + 1e-5)
    y = xhat * cst_ref[2:3, :] + cst_ref[3:4, :]
    o2 = jnp.maximum(y, 0.0) + jnp.log(1.0 + jnp.exp(-jnp.abs(y)))
    ot = jnp.stack([jnp.transpose(o2[j * bn:(j + 1) * bn]) for j in range(m)],
                   axis=0)
    out_ref[...] = ot.reshape(m, at_ref.shape[1], at_ref.shape[2], bn)


def _alias_body(inner, prev_ref, *refs):
    del prev_ref
    inner(*refs)


def _sc_gather_chunk(table, idx2, c):
    rows = _JC * _N
    off = c * rows // _GW
    mesh = plsc.VectorSubcoreMesh(core_axis_name="core",
                                  subcore_axis_name="subcore")

    @pl.kernel(
        out_type=jax.ShapeDtypeStruct((rows, _MP), jnp.float32),
        mesh=mesh)
    def gather_kernel(t_hbm, i_hbm, g_hbm):
        def body(i_vmem, g_vmem):
            pltpu.sync_copy(t_hbm.at[i_vmem.at[0]], g_vmem)

        pltpu.emit_pipeline(
            body,
            grid=(rows // _GW,),
            in_specs=[pl.BlockSpec((1, _GW), lambda i: (0, off + i))],
            out_specs=[pl.BlockSpec((_GW, _MP), lambda i: (i, 0))],
            core_axis_name=("core", "subcore"),
            dimension_semantics=(pltpu.PARALLEL,),
        )(i_hbm, g_hbm)

    return gather_kernel(table, idx2)


def kernel(angle_fea, edge_fea, nbr_idx, W_att, b_att, W_lin, b_lin, gamma,
           beta):
    f32 = jnp.float32
    eye = jnp.eye(_M, dtype=f32)
    W1, W2, W3 = W_lin[:_P], W_lin[_P:2 * _P], W_lin[2 * _P:]
    w1, w2, w3 = W_att[:_P, 0], W_att[_P:2 * _P, 0], W_att[2 * _P:, 0]

    bd3 = jnp.kron(eye, W3)
    bdv3 = jnp.kron(eye, jnp.tile(w3[:, None], (1, _P)))
    waf = jnp.concatenate([bd3, bdv3], axis=1)
    bd2 = jnp.kron(eye, W2)
    bdv2 = jnp.kron(eye, jnp.tile(w2[:, None], (1, _P)))
    wg = jnp.concatenate([bd2, bdv2], axis=1)
    w1t = jnp.tile(W1, (1, _M))
    ones16 = jnp.kron(eye, jnp.full((_P, _P), 1.0 / _P, f32))
    consts = jnp.stack([
        jnp.tile(b_lin, _M),
        jnp.full((_MP,), b_att[0], f32),
        jnp.tile(gamma, _M),
        jnp.tile(beta, _M),
        jnp.concatenate([w1, jnp.zeros((_MP - _P,), f32)]),
    ])

    angle_t = jnp.transpose(angle_fea, (1, 2, 3, 0))
    edge_t = jnp.transpose(edge_fea, (1, 2, 0))
    idx2 = jnp.transpose(nbr_idx, (1, 0)).reshape(1, _R)

    edge2 = pl.pallas_call(
        _tr_edge_body,
        grid=(-(-_N // _EB),),
        in_specs=[pl.BlockSpec((_M, _P, _EB), lambda i: (0, 0, i))],
        out_specs=pl.BlockSpec((_EB, _MP), lambda i: (i, 0)),
        out_shape=jax.ShapeDtypeStruct((_N, _MP), f32),
        compiler_params=pltpu.CompilerParams(
            dimension_semantics=("parallel",)),
    )(edge_t)

    ges = [_sc_gather_chunk(edge2, idx2, c).reshape(_JC, _N, _MP)
           for c in range(_NC)]

    weight_specs = [
        pl.BlockSpec((_MP, 2 * _MP), lambda i: (0, 0)),
        pl.BlockSpec((_MP, 2 * _MP), lambda i: (0, 0)),
        pl.BlockSpec((_P, _MP), lambda i: (0, 0)),
        pl.BlockSpec((_MP, _MP), lambda i: (0, 0)),
        pl.BlockSpec((5, _MP), lambda i: (0, 0)),
    ]
    out_t = None
    for c in range(_NC):
        data_specs = [
            pl.BlockSpec((_JC, _P, _P, _BN),
                         functools.partial(lambda c, i: (c, 0, 0, i), c)),
            pl.BlockSpec((_JC, _BN, _MP), lambda i: (0, i, 0)),
            pl.BlockSpec((_BN, _MP), lambda i: (i, 0)),
        ]
        out_spec = pl.BlockSpec((_JC, _P, _P, _BN),
                                functools.partial(lambda c, i: (c, 0, 0, i), c))
        body = functools.partial(_main_body, j_off=c * _JC)
        if out_t is None:
            out_t = pl.pallas_call(
                body,
                grid=(_NB,),
                in_specs=data_specs + weight_specs,
                out_specs=out_spec,
                out_shape=jax.ShapeDtypeStruct((_M, _P, _P, _N), f32),
                compiler_params=pltpu.CompilerParams(
                    dimension_semantics=("parallel",)),
            )(angle_t, ges[c], edge2, waf, wg, w1t, ones16, consts)
        else:
            out_t = pl.pallas_call(
                functools.partial(_alias_body, body),
                grid=(_NB,),
                in_specs=[pl.BlockSpec(memory_space=pl.ANY)] + data_specs
                + weight_specs,
                out_specs=out_spec,
                out_shape=jax.ShapeDtypeStruct((_M, _P, _P, _N), f32),
                input_output_aliases={0: 0},
                compiler_params=pltpu.CompilerParams(
                    dimension_semantics=("parallel",)),
            )(out_t, angle_t, ges[c], edge2, waf, wg, w1t, ones16, consts)

    return jnp.transpose(out_t, (3, 0, 1, 2))


# speedup vs baseline: 8.5100x; 1.0335x over previous
import functools

import jax
import jax.numpy as jnp
from jax import lax
from jax.experimental import pallas as pl
from jax.experimental.pallas import tpu as pltpu
from jax.experimental.pallas import tpu_sc as plsc

_N, _M, _P = 10000, 16, 16
_MP = _M * _P
_R = _N * _M
_GW = 128
_NC = 2
_JC = _M // _NC
_BN = 512
_NB = -(-_N // _BN)
_EB = 2048


def _tr_edge_body(et_ref, e2_ref):
    s = et_ref.shape
    x = et_ref[...].reshape(s[0] * s[1], s[2])
    e2_ref[...] = jnp.transpose(x)


def _main_body(at_ref, ge_ref, et_ref, waf_ref, wg_ref, w1t_ref, ones_ref,
               cst_ref, out_ref):
    bn = at_ref.shape[3]
    m = at_ref.shape[0]
    mp = at_ref.shape[1] * at_ref.shape[2]
    a2 = at_ref[...].reshape(m, mp, bn)
    af = jnp.concatenate([jnp.transpose(a2[j]) for j in range(m)], axis=0)
    ge = ge_ref[...].reshape(m * bn, mp)
    et = et_ref[...]
    e3 = jnp.concatenate([jnp.transpose(et[j]) for j in range(m)],
                         axis=0)

    xaf = jnp.dot(af, waf_ref[...], preferred_element_type=jnp.float32)
    xg = jnp.dot(ge, wg_ref[...], preferred_element_type=jnp.float32)
    ut = jnp.dot(e3, w1t_ref[...], preferred_element_type=jnp.float32)
    a_col = jnp.sum(e3 * cst_ref[4:5, :16], axis=1, keepdims=True)

    lin = xaf[:, :mp] + xg[:, :mp] + ut + cst_ref[0:1, :]
    att = xaf[:, mp:] + xg[:, mp:] + a_col + cst_ref[1:2, :]
    alpha = jnp.where(att >= 0, att, 0.01 * att)
    s = af + alpha * lin
    ones16 = ones_ref[...]
    mu = jnp.dot(s, ones16, preferred_element_type=jnp.float32)
    d = s - mu
    var = jnp.dot(d * d, ones16, preferred_element_type=jnp.float32)
    xhat = d * lax.rsqrt(var + 1e-5)
    y = xhat * cst_ref[2:3, :] + cst_ref[3:4, :]
    o2 = jnp.maximum(y, 0.0) + jnp.log(1.0 + jnp.exp(-jnp.abs(y)))
    ot = jnp.stack([jnp.transpose(o2[j * bn:(j + 1) * bn]) for j in range(m)],
                   axis=0)
    out_ref[...] = ot.reshape(m, at_ref.shape[1], at_ref.shape[2], bn)


def _alias_body(inner, prev_ref, *refs):
    del prev_ref
    inner(*refs)


def _sc_gather_chunk(table, idx2, c):
    rows = _JC * _N
    off = c * rows // _GW
    mesh = plsc.VectorSubcoreMesh(core_axis_name="core",
                                  subcore_axis_name="subcore")

    @pl.kernel(
        out_type=jax.ShapeDtypeStruct((rows, _MP), jnp.float32),
        mesh=mesh)
    def gather_kernel(t_hbm, i_hbm, g_hbm):
        def body(i_vmem, g_vmem):
            pltpu.sync_copy(t_hbm.at[i_vmem.at[0]], g_vmem)

        pltpu.emit_pipeline(
            body,
            grid=(rows // _GW,),
            in_specs=[pl.BlockSpec((1, _GW), lambda i: (0, off + i))],
            out_specs=[pl.BlockSpec((_GW, _MP), lambda i: (i, 0))],
            core_axis_name=("core", "subcore"),
            dimension_semantics=(pltpu.PARALLEL,),
        )(i_hbm, g_hbm)

    return gather_kernel(table, idx2)


def kernel(angle_fea, edge_fea, nbr_idx, W_att, b_att, W_lin, b_lin, gamma,
           beta):
    f32 = jnp.float32
    eye = jnp.eye(_M, dtype=f32)
    W1, W2, W3 = W_lin[:_P], W_lin[_P:2 * _P], W_lin[2 * _P:]
    w1, w2, w3 = W_att[:_P, 0], W_att[_P:2 * _P, 0], W_att[2 * _P:, 0]

    bd3 = jnp.kron(eye, W3)
    bdv3 = jnp.kron(eye, jnp.tile(w3[:, None], (1, _P)))
    waf = jnp.concatenate([bd3, bdv3], axis=1)
    bd2 = jnp.kron(eye, W2)
    bdv2 = jnp.kron(eye, jnp.tile(w2[:, None], (1, _P)))
    wg = jnp.concatenate([bd2, bdv2], axis=1)
    w1t = jnp.tile(W1, (1, _M))
    ones16 = jnp.kron(eye, jnp.full((_P, _P), 1.0 / _P, f32))
    consts = jnp.stack([
        jnp.tile(b_lin, _M),
        jnp.full((_MP,), b_att[0], f32),
        jnp.tile(gamma, _M),
        jnp.tile(beta, _M),
        jnp.concatenate([w1, jnp.zeros((_MP - _P,), f32)]),
    ])

    angle_t = jnp.transpose(angle_fea, (1, 2, 3, 0))
    edge_t = jnp.transpose(edge_fea, (1, 2, 0))
    idx2 = jnp.transpose(nbr_idx, (1, 0)).reshape(1, _R)

    edge2 = pl.pallas_call(
        _tr_edge_body,
        grid=(-(-_N // _EB),),
        in_specs=[pl.BlockSpec((_M, _P, _EB), lambda i: (0, 0, i))],
        out_specs=pl.BlockSpec((_EB, _MP), lambda i: (i, 0)),
        out_shape=jax.ShapeDtypeStruct((_N, _MP), f32),
        compiler_params=pltpu.CompilerParams(
            dimension_semantics=("parallel",)),
    )(edge_t)

    ges = [_sc_gather_chunk(edge2, idx2, c).reshape(_JC, _N, _MP)
           for c in range(_NC)]

    weight_specs = [
        pl.BlockSpec((_MP, 2 * _MP), lambda i: (0, 0)),
        pl.BlockSpec((_MP, 2 * _MP), lambda i: (0, 0)),
        pl.BlockSpec((_P, _MP), lambda i: (0, 0)),
        pl.BlockSpec((_MP, _MP), lambda i: (0, 0)),
        pl.BlockSpec((5, _MP), lambda i: (0, 0)),
    ]
    out_t = None
    for c in range(_NC):
        data_specs = [
            pl.BlockSpec((_JC, _P, _P, _BN),
                         functools.partial(lambda c, i: (c, 0, 0, i), c)),
            pl.BlockSpec((_JC, _BN, _MP), lambda i: (0, i, 0)),
            pl.BlockSpec((_JC, _P, _BN),
                         functools.partial(lambda c, i: (c, 0, i), c)),
        ]
        out_spec = pl.BlockSpec((_JC, _P, _P, _BN),
                                functools.partial(lambda c, i: (c, 0, 0, i), c))
        if out_t is None:
            out_t = pl.pallas_call(
                _main_body,
                grid=(_NB,),
                in_specs=data_specs + weight_specs,
                out_specs=out_spec,
                out_shape=jax.ShapeDtypeStruct((_M, _P, _P, _N), f32),
                compiler_params=pltpu.CompilerParams(
                    dimension_semantics=("parallel",)),
            )(angle_t, ges[c], edge_t, waf, wg, w1t, ones16, consts)
        else:
            out_t = pl.pallas_call(
                functools.partial(_alias_body, _main_body),
                grid=(_NB,),
                in_specs=[pl.BlockSpec(memory_space=pl.ANY)] + data_specs
                + weight_specs,
                out_specs=out_spec,
                out_shape=jax.ShapeDtypeStruct((_M, _P, _P, _N), f32),
                input_output_aliases={0: 0},
                compiler_params=pltpu.CompilerParams(
                    dimension_semantics=("parallel",)),
            )(out_t, angle_t, ges[c], edge_t, waf, wg, w1t, ones16, consts)

    return jnp.transpose(out_t, (3, 0, 1, 2))


# speedup vs baseline: 9.4932x; 1.1155x over previous
import functools

import jax
import jax.numpy as jnp
from jax import lax
from jax.experimental import pallas as pl
from jax.experimental.pallas import tpu as pltpu
from jax.experimental.pallas import tpu_sc as plsc

_N, _M, _P = 10000, 16, 16
_MP = _M * _P
_R = _N * _M
_GW = 128
_NC = 2
_JC = _M // _NC
_BN = 256
_NB = -(-_N // _BN)
_EB = 2048


def _tr_edge_body(et_ref, e2_ref):
    s = et_ref.shape
    x = et_ref[...].reshape(s[0] * s[1], s[2])
    t = jnp.transpose(x)
    h = t.shape[1] // 2
    a = lax.bitcast_convert_type(t[:, :h], jnp.int32)
    b = lax.bitcast_convert_type(t[:, h:], jnp.int32)
    lo = jnp.bitwise_and(jnp.right_shift(a + 32768, 16), jnp.int32(0xFFFF))
    hi = jnp.bitwise_and(b + 32768, jnp.int32(-65536))
    e2_ref[...] = jnp.bitwise_or(lo, hi)


def _main_body(at_ref, ge_ref, et_ref, waf_ref, wg_ref, w1t_ref, ones_ref,
               cst_ref, out_ref):
    bn = at_ref.shape[3]
    m = at_ref.shape[0]
    mp = at_ref.shape[1] * at_ref.shape[2]
    a2 = at_ref[...].reshape(m, mp, bn)
    af = jnp.concatenate([jnp.transpose(a2[j]) for j in range(m)], axis=0)
    gr = ge_ref[...].reshape(m * bn, mp // 2)
    glo = lax.bitcast_convert_type(jnp.left_shift(gr, 16), jnp.float32)
    ghi = lax.bitcast_convert_type(
        jnp.bitwise_and(gr, jnp.int32(-65536)), jnp.float32)
    ge = jnp.concatenate([glo, ghi], axis=1)
    et = et_ref[...]
    e3 = jnp.concatenate([jnp.transpose(et[j]) for j in range(m)],
                         axis=0)

    xaf = jnp.dot(af, waf_ref[...], preferred_element_type=jnp.float32)
    xg = jnp.dot(ge, wg_ref[...], preferred_element_type=jnp.float32)
    ut = jnp.dot(e3, w1t_ref[...], preferred_element_type=jnp.float32)
    a_col = jnp.sum(e3 * cst_ref[4:5, :16], axis=1, keepdims=True)

    lin = xaf[:, :mp] + xg[:, :mp] + ut + cst_ref[0:1, :]
    att = xaf[:, mp:] + xg[:, mp:] + a_col + cst_ref[1:2, :]
    alpha = jnp.where(att >= 0, att, 0.01 * att)
    s = af + alpha * lin
    ones16 = ones_ref[...]
    mu = jnp.dot(s, ones16, preferred_element_type=jnp.float32)
    d = s - mu
    var = jnp.dot(d * d, ones16, preferred_element_type=jnp.float32)
    xhat = d * lax.rsqrt(var + 1e-5)
    y = xhat * cst_ref[2:3, :] + cst_ref[3:4, :]
    o2 = jnp.maximum(y, 0.0) + jnp.log(1.0 + jnp.exp(-jnp.abs(y)))
    ot = jnp.stack([jnp.transpose(o2[j * bn:(j + 1) * bn]) for j in range(m)],
                   axis=0)
    out_ref[...] = ot.reshape(m, at_ref.shape[1], at_ref.shape[2], bn)


def _alias_body(inner, prev_ref, *refs):
    del prev_ref
    inner(*refs)


def _sc_gather_chunk(table, idx2, c):
    rows = _JC * _N
    off = c * rows // _GW
    mesh = plsc.VectorSubcoreMesh(core_axis_name="core",
                                  subcore_axis_name="subcore")

    @pl.kernel(
        out_type=jax.ShapeDtypeStruct((rows, _MP // 2), jnp.int32),
        mesh=mesh)
    def gather_kernel(t_hbm, i_hbm, g_hbm):
        def body(i_vmem, g_vmem):
            pltpu.sync_copy(t_hbm.at[i_vmem.at[0]], g_vmem)

        pltpu.emit_pipeline(
            body,
            grid=(rows // _GW,),
            in_specs=[pl.BlockSpec((1, _GW), lambda i: (0, off + i))],
            out_specs=[pl.BlockSpec((_GW, _MP // 2), lambda i: (i, 0))],
            core_axis_name=("core", "subcore"),
            dimension_semantics=(pltpu.PARALLEL,),
        )(i_hbm, g_hbm)

    return gather_kernel(table, idx2)


def kernel(angle_fea, edge_fea, nbr_idx, W_att, b_att, W_lin, b_lin, gamma,
           beta):
    f32 = jnp.float32
    eye = jnp.eye(_M, dtype=f32)
    W1, W2, W3 = W_lin[:_P], W_lin[_P:2 * _P], W_lin[2 * _P:]
    w1, w2, w3 = W_att[:_P, 0], W_att[_P:2 * _P, 0], W_att[2 * _P:, 0]

    bd3 = jnp.kron(eye, W3)
    bdv3 = jnp.kron(eye, jnp.tile(w3[:, None], (1, _P)))
    waf = jnp.concatenate([bd3, bdv3], axis=1)
    bd2 = jnp.kron(eye, W2)
    bdv2 = jnp.kron(eye, jnp.tile(w2[:, None], (1, _P)))
    wg = jnp.concatenate([bd2, bdv2], axis=1)
    w1t = jnp.tile(W1, (1, _M))
    ones16 = jnp.kron(eye, jnp.full((_P, _P), 1.0 / _P, f32))
    consts = jnp.stack([
        jnp.tile(b_lin, _M),
        jnp.full((_MP,), b_att[0], f32),
        jnp.tile(gamma, _M),
        jnp.tile(beta, _M),
        jnp.concatenate([w1, jnp.zeros((_MP - _P,), f32)]),
    ])

    angle_t = jnp.transpose(angle_fea, (1, 2, 3, 0))
    edge_t = jnp.transpose(edge_fea, (1, 2, 0))
    idx2 = jnp.transpose(nbr_idx, (1, 0)).reshape(1, _R)

    edge2 = pl.pallas_call(
        _tr_edge_body,
        grid=(-(-_N // _EB),),
        in_specs=[pl.BlockSpec((_M, _P, _EB), lambda i: (0, 0, i))],
        out_specs=pl.BlockSpec((_EB, _MP // 2), lambda i: (i, 0)),
        out_shape=jax.ShapeDtypeStruct((_N, _MP // 2), jnp.int32),
        compiler_params=pltpu.CompilerParams(
            dimension_semantics=("parallel",)),
    )(edge_t)

    ges = [_sc_gather_chunk(edge2, idx2, c).reshape(_JC, _N, _MP // 2)
           for c in range(_NC)]

    weight_specs = [
        pl.BlockSpec((_MP, 2 * _MP), lambda i: (0, 0)),
        pl.BlockSpec((_MP, 2 * _MP), lambda i: (0, 0)),
        pl.BlockSpec((_P, _MP), lambda i: (0, 0)),
        pl.BlockSpec((_MP, _MP), lambda i: (0, 0)),
        pl.BlockSpec((5, _MP), lambda i: (0, 0)),
    ]
    out_t = None
    for c in range(_NC):
        data_specs = [
            pl.BlockSpec((_JC, _P, _P, _BN),
                         functools.partial(lambda c, i: (c, 0, 0, i), c)),
            pl.BlockSpec((_JC, _BN, _MP // 2), lambda i: (0, i, 0)),
            pl.BlockSpec((_JC, _P, _BN),
                         functools.partial(lambda c, i: (c, 0, i), c)),
        ]
        out_spec = pl.BlockSpec((_JC, _P, _P, _BN),
                                functools.partial(lambda c, i: (c, 0, 0, i), c))
        if out_t is None:
            out_t = pl.pallas_call(
                _main_body,
                grid=(_NB,),
                in_specs=data_specs + weight_specs,
                out_specs=out_spec,
                out_shape=jax.ShapeDtypeStruct((_M, _P, _P, _N), f32),
                compiler_params=pltpu.CompilerParams(
                    dimension_semantics=("parallel",)),
            )(angle_t, ges[c], edge_t, waf, wg, w1t, ones16, consts)
        else:
            out_t = pl.pallas_call(
                functools.partial(_alias_body, _main_body),
                grid=(_NB,),
                in_specs=[pl.BlockSpec(memory_space=pl.ANY)] + data_specs
                + weight_specs,
                out_specs=out_spec,
                out_shape=jax.ShapeDtypeStruct((_M, _P, _P, _N), f32),
                input_output_aliases={0: 0},
                compiler_params=pltpu.CompilerParams(
                    dimension_semantics=("parallel",)),
            )(out_t, angle_t, ges[c], edge_t, waf, wg, w1t, ones16, consts)

    return jnp.transpose(out_t, (3, 0, 1, 2))


# speedup vs baseline: 9.7353x; 1.0255x over previous
import functools

import jax
import jax.numpy as jnp
from jax import lax
from jax.experimental import pallas as pl
from jax.experimental.pallas import tpu as pltpu
from jax.experimental.pallas import tpu_sc as plsc

_N, _M, _P = 10000, 16, 16
_MP = _M * _P
_R = _N * _M
_GW = 128
_NC = 2
_JC = _M // _NC
_BN = 512
_NB = -(-_N // _BN)
_EB = 2048


def _tr_edge_body(et_ref, e2_ref):
    s = et_ref.shape
    x = et_ref[...].reshape(s[0] * s[1], s[2])
    t = jnp.transpose(x)
    h = t.shape[1] // 2
    a = lax.bitcast_convert_type(t[:, :h], jnp.int32)
    b = lax.bitcast_convert_type(t[:, h:], jnp.int32)
    lo = jnp.bitwise_and(jnp.right_shift(a + 32768, 16), jnp.int32(0xFFFF))
    hi = jnp.bitwise_and(b + 32768, jnp.int32(-65536))
    e2_ref[...] = jnp.bitwise_or(lo, hi)


def _main_body(at_ref, ge_ref, et_ref, waf_ref, wg_ref, w1t_ref, ones_ref,
               cst_ref, out_ref):
    bn = at_ref.shape[3]
    m = at_ref.shape[0]
    mp = at_ref.shape[1] * at_ref.shape[2]
    a2 = at_ref[...].reshape(m, mp, bn)
    af = jnp.concatenate([jnp.transpose(a2[j]) for j in range(m)], axis=0)
    gr = ge_ref[...].reshape(m * bn, mp // 2)
    glo = lax.bitcast_convert_type(jnp.left_shift(gr, 16), jnp.float32)
    ghi = lax.bitcast_convert_type(
        jnp.bitwise_and(gr, jnp.int32(-65536)), jnp.float32)
    ge = jnp.concatenate([glo, ghi], axis=1)
    et = et_ref[...]
    e3 = jnp.concatenate([jnp.transpose(et[j]) for j in range(m)],
                         axis=0)

    xaf = jnp.dot(af, waf_ref[...], preferred_element_type=jnp.float32)
    xg = jnp.dot(ge, wg_ref[...], preferred_element_type=jnp.float32)
    ut = jnp.dot(e3, w1t_ref[...], preferred_element_type=jnp.float32)
    a_col = jnp.sum(e3 * cst_ref[4:5, :16], axis=1, keepdims=True)

    lin = xaf[:, :mp] + xg[:, :mp] + ut + cst_ref[0:1, :]
    att = xaf[:, mp:] + xg[:, mp:] + a_col + cst_ref[1:2, :]
    alpha = jnp.where(att >= 0, att, 0.01 * att)
    s = af + alpha * lin
    ones16 = ones_ref[...]
    mu = jnp.dot(s, ones16, preferred_element_type=jnp.float32)
    d = s - mu
    var = jnp.dot(d * d, ones16, preferred_element_type=jnp.float32)
    xhat = d * lax.rsqrt(var + 1e-5)
    y = xhat * cst_ref[2:3, :] + cst_ref[3:4, :]
    o2 = jnp.maximum(y, 0.0) + jnp.log(1.0 + jnp.exp(-jnp.abs(y)))
    ot = jnp.stack([jnp.transpose(o2[j * bn:(j + 1) * bn]) for j in range(m)],
                   axis=0)
    out_ref[...] = ot.reshape(m, at_ref.shape[1], at_ref.shape[2], bn)


def _alias_body(inner, prev_ref, *refs):
    del prev_ref
    inner(*refs)


def _sc_gather_chunk(table, idx2, c):
    rows = _JC * _N
    off = c * rows // _GW
    mesh = plsc.VectorSubcoreMesh(core_axis_name="core",
                                  subcore_axis_name="subcore")

    @pl.kernel(
        out_type=jax.ShapeDtypeStruct((rows, _MP // 2), jnp.int32),
        mesh=mesh)
    def gather_kernel(t_hbm, i_hbm, g_hbm):
        def body(i_vmem, g_vmem):
            pltpu.sync_copy(t_hbm.at[i_vmem.at[0]], g_vmem)

        pltpu.emit_pipeline(
            body,
            grid=(rows // _GW,),
            in_specs=[pl.BlockSpec((1, _GW), lambda i: (0, off + i))],
            out_specs=[pl.BlockSpec((_GW, _MP // 2), lambda i: (i, 0))],
            core_axis_name=("core", "subcore"),
            dimension_semantics=(pltpu.PARALLEL,),
        )(i_hbm, g_hbm)

    return gather_kernel(table, idx2)


def kernel(angle_fea, edge_fea, nbr_idx, W_att, b_att, W_lin, b_lin, gamma,
           beta):
    f32 = jnp.float32
    eye = jnp.eye(_M, dtype=f32)
    W1, W2, W3 = W_lin[:_P], W_lin[_P:2 * _P], W_lin[2 * _P:]
    w1, w2, w3 = W_att[:_P, 0], W_att[_P:2 * _P, 0], W_att[2 * _P:, 0]

    bd3 = jnp.kron(eye, W3)
    bdv3 = jnp.kron(eye, jnp.tile(w3[:, None], (1, _P)))
    waf = jnp.concatenate([bd3, bdv3], axis=1)
    bd2 = jnp.kron(eye, W2)
    bdv2 = jnp.kron(eye, jnp.tile(w2[:, None], (1, _P)))
    wg = jnp.concatenate([bd2, bdv2], axis=1)
    w1t = jnp.tile(W1, (1, _M))
    ones16 = jnp.kron(eye, jnp.full((_P, _P), 1.0 / _P, f32))
    consts = jnp.stack([
        jnp.tile(b_lin, _M),
        jnp.full((_MP,), b_att[0], f32),
        jnp.tile(gamma, _M),
        jnp.tile(beta, _M),
        jnp.concatenate([w1, jnp.zeros((_MP - _P,), f32)]),
    ])

    angle_t = jnp.transpose(angle_fea, (1, 2, 3, 0))
    edge_t = jnp.transpose(edge_fea, (1, 2, 0))
    idx2 = jnp.transpose(nbr_idx, (1, 0)).reshape(1, _R)

    edge2 = pl.pallas_call(
        _tr_edge_body,
        grid=(-(-_N // _EB),),
        in_specs=[pl.BlockSpec((_M, _P, _EB), lambda i: (0, 0, i))],
        out_specs=pl.BlockSpec((_EB, _MP // 2), lambda i: (i, 0)),
        out_shape=jax.ShapeDtypeStruct((_N, _MP // 2), jnp.int32),
        compiler_params=pltpu.CompilerParams(
            dimension_semantics=("parallel",)),
    )(edge_t)

    ges = [_sc_gather_chunk(edge2, idx2, c).reshape(_JC, _N, _MP // 2)
           for c in range(_NC)]

    weight_specs = [
        pl.BlockSpec((_MP, 2 * _MP), lambda i: (0, 0)),
        pl.BlockSpec((_MP, 2 * _MP), lambda i: (0, 0)),
        pl.BlockSpec((_P, _MP), lambda i: (0, 0)),
        pl.BlockSpec((_MP, _MP), lambda i: (0, 0)),
        pl.BlockSpec((5, _MP), lambda i: (0, 0)),
    ]
    out_t = None
    for c in range(_NC):
        data_specs = [
            pl.BlockSpec((_JC, _P, _P, _BN),
                         functools.partial(lambda c, i: (c, 0, 0, i), c)),
            pl.BlockSpec((_JC, _BN, _MP // 2), lambda i: (0, i, 0)),
            pl.BlockSpec((_JC, _P, _BN),
                         functools.partial(lambda c, i: (c, 0, i), c)),
        ]
        out_spec = pl.BlockSpec((_JC, _P, _P, _BN),
                                functools.partial(lambda c, i: (c, 0, 0, i), c))
        if out_t is None:
            out_t = pl.pallas_call(
                _main_body,
                grid=(_NB,),
                in_specs=data_specs + weight_specs,
                out_specs=out_spec,
                out_shape=jax.ShapeDtypeStruct((_M, _P, _P, _N), f32),
                compiler_params=pltpu.CompilerParams(
                    dimension_semantics=("parallel",),
                    vmem_limit_bytes=50 * 1024 * 1024),
            )(angle_t, ges[c], edge_t, waf, wg, w1t, ones16, consts)
        else:
            out_t = pl.pallas_call(
                functools.partial(_alias_body, _main_body),
                grid=(_NB,),
                in_specs=[pl.BlockSpec(memory_space=pl.ANY)] + data_specs
                + weight_specs,
                out_specs=out_spec,
                out_shape=jax.ShapeDtypeStruct((_M, _P, _P, _N), f32),
                input_output_aliases={0: 0},
                compiler_params=pltpu.CompilerParams(
                    dimension_semantics=("parallel",),
                    vmem_limit_bytes=50 * 1024 * 1024),
            )(out_t, angle_t, ges[c], edge_t, waf, wg, w1t, ones16, consts)

    return jnp.transpose(out_t, (3, 0, 1, 2))
